# baseline (speedup 1.0000x reference)
import functools

import jax
import jax.numpy as jnp
from jax import lax
from jax.experimental import pallas as pl
from jax.experimental.pallas import tpu as pltpu
from jax.experimental.pallas import tpu_sc as plsc

N = 10000
D = 128
NC = 2
NS = 16
NW = NC * NS
GROUP = 128
R = 10240
RPT = R // NS
BR = 1000

_MESH = dict(mesh=plsc.VectorSubcoreMesh(core_axis_name="c", subcore_axis_name="s"))


def _deg_body(dst_hbm, ones_hbm, zero_hbm, out_hbm, dst_v, ones_v, deg_sh, sem):
    c = lax.axis_index("c")
    s = lax.axis_index("s")
    wid = c * NS + s
    kc = KA if KA == KB else jnp.where(c == 0, KA, KB)
    pltpu.sync_copy(ones_hbm, ones_v)
    pltpu.sync_copy(zero_hbm, deg_sh.at[pl.ds(s * RPT, RPT)])
    pltpu.sync_copy(dst_hbm.at[wid], dst_v)
    plsc.subcore_barrier()

    def issue(j, carry):
        pltpu.async_copy(ones_v, deg_sh.at[dst_v.at[j]], sem, add=True)
        return carry

    lax.fori_loop(0, kc, issue, 0)

    def drain(j, carry):
        pltpu.make_async_copy(ones_v, deg_sh.at[dst_v.at[0]], sem).wait()
        return carry

    lax.fori_loop(0, kc, drain, 0)
    plsc.subcore_barrier()
    pltpu.sync_copy(deg_sh.at[pl.ds(s * RPT, RPT)], out_hbm.at[c, pl.ds(s * RPT, RPT)])


def _sc_degree(dst_p):
    ones = jnp.ones((GROUP,), jnp.float32)
    zero = jnp.zeros((RPT,), jnp.float32)
    return pl.kernel(
        _deg_body,
        out_type=jax.ShapeDtypeStruct((NC, R), jnp.float32),
        scratch_types=[
            pltpu.VMEM((KMAX, GROUP), jnp.int32),
            pltpu.VMEM((GROUP,), jnp.float32),
            pltpu.VMEM_SHARED((R,), jnp.float32),
            pltpu.SemaphoreType.DMA,
        ],
        **_MESH,
    )(dst_p, ones, zero)


KA = 80
KB = 80
KMAX = max(KA, KB)


CG = 16
NCHUNK = 5


def _scat_body(hp_hbm, src_hbm, dst_hbm, zeros_hbm, out_hbm,
               srcc, dstc, rows0, rows1, out_sh,
               isem, gsem0, gsem1, ssem0, ssem1):
    c = lax.axis_index("c")
    s = lax.axis_index("s")
    wid = c * NS + s
    i0 = pltpu.async_copy(src_hbm.at[wid, pl.ds(0, CG)], srcc.at[0], isem)
    i1 = pltpu.async_copy(dst_hbm.at[wid, pl.ds(0, CG)], dstc.at[0], isem)
    pltpu.sync_copy(zeros_hbm, out_sh.at[pl.ds(s * RPT, RPT)])
    i0.wait()
    i1.wait()
    plsc.subcore_barrier()

    for t in range(NCHUNK):
        cb = t % 2
        if t + 1 < NCHUNK:
            pltpu.async_copy(src_hbm.at[wid, pl.ds((t + 1) * CG, CG)],
                             srcc.at[1 - cb], isem)
            pltpu.async_copy(dst_hbm.at[wid, pl.ds((t + 1) * CG, CG)],
                             dstc.at[1 - cb], isem)

        pltpu.async_copy(hp_hbm.at[srcc.at[cb, 0]], rows0, gsem0)

        def pair(p, carry):
            pltpu.make_async_copy(hp_hbm.at[srcc.at[cb, 2 * p]], rows0,
                                  gsem0).wait()
            pltpu.async_copy(hp_hbm.at[srcc.at[cb, 2 * p + 1]], rows1, gsem1)
            s0 = pltpu.async_copy(rows0, out_sh.at[dstc.at[cb, 2 * p]],
                                  ssem0, add=True)
            pltpu.make_async_copy(hp_hbm.at[srcc.at[cb, 2 * p + 1]], rows1,
                                  gsem1).wait()
            s1 = pltpu.async_copy(rows1, out_sh.at[dstc.at[cb, 2 * p + 1]],
                                  ssem1, add=True)
            s0.wait()
            nxt = lax.rem(2 * p + 2, CG)
            pltpu.async_copy(hp_hbm.at[srcc.at[cb, nxt]], rows0, gsem0)
            s1.wait()
            return carry

        lax.fori_loop(0, CG // 2, pair, 0)
        pltpu.make_async_copy(hp_hbm.at[srcc.at[cb, 0]], rows0, gsem0).wait()
        if t + 1 < NCHUNK:
            pltpu.make_async_copy(src_hbm.at[wid, pl.ds(0, CG)],
                                  srcc.at[1 - cb], isem).wait()
            pltpu.make_async_copy(dst_hbm.at[wid, pl.ds(0, CG)],
                                  dstc.at[1 - cb], isem).wait()

    plsc.subcore_barrier()
    pltpu.sync_copy(out_sh.at[pl.ds(s * RPT, RPT)], out_hbm.at[c, pl.ds(s * RPT, RPT)])


def _sc_scatter(hp, src_p, dst_p):
    zeros = jnp.zeros((RPT, D), jnp.float32)
    return pl.kernel(
        _scat_body,
        out_type=jax.ShapeDtypeStruct((NC, R, D), jnp.float32),
        scratch_types=[
            pltpu.VMEM((2, CG, GROUP), jnp.int32),
            pltpu.VMEM((2, CG, GROUP), jnp.int32),
            pltpu.VMEM((GROUP, D), jnp.float32),
            pltpu.VMEM((GROUP, D), jnp.float32),
            pltpu.VMEM_SHARED((R, D), jnp.float32),
            pltpu.SemaphoreType.DMA,
            pltpu.SemaphoreType.DMA,
            pltpu.SemaphoreType.DMA,
            pltpu.SemaphoreType.DMA,
            pltpu.SemaphoreType.DMA,
        ],
        **_MESH,
    )(hp, src_p, dst_p, zeros)


def _tcmm_body(x_ref, w_ref, m_ref):
    m_ref[...] = jnp.dot(x_ref[...], w_ref[...],
                         preferred_element_type=jnp.float32)


def _tcmm(x, w):
    grid = (N // BR,)
    return pl.pallas_call(
        _tcmm_body,
        grid=grid,
        in_specs=[
            pl.BlockSpec((BR, D), lambda i: (i, 0)),
            pl.BlockSpec((D, D), lambda i: (0, 0)),
        ],
        out_specs=pl.BlockSpec((BR, D), lambda i: (i, 0)),
        out_shape=jax.ShapeDtypeStruct((N, D), jnp.float32),
    )(x, w)


def _tc1_body(dega_ref, degb_ref, m_ref, dis_ref, hp_ref):
    deg = dega_ref[...] + degb_ref[...] + 1.0
    dis = lax.rsqrt(deg)
    dis_ref[...] = dis
    hp_ref[...] = m_ref[...] * dis


def _tc1(dega, degb, m):
    grid = (N // BR,)
    return pl.pallas_call(
        _tc1_body,
        grid=grid,
        in_specs=[
            pl.BlockSpec((BR, 1), lambda i: (i, 0)),
            pl.BlockSpec((BR, 1), lambda i: (i, 0)),
            pl.BlockSpec((BR, D), lambda i: (i, 0)),
        ],
        out_specs=[
            pl.BlockSpec((BR, 1), lambda i: (i, 0)),
            pl.BlockSpec((BR, D), lambda i: (i, 0)),
        ],
        out_shape=[
            jax.ShapeDtypeStruct((N, 1), jnp.float32),
            jax.ShapeDtypeStruct((N, D), jnp.float32),
        ],
    )(dega, degb, m)


def _tc2_body(sa_ref, sb_ref, hp_ref, dis_ref, b_ref, w_ref, out_ref):
    dis = dis_ref[...]
    h = (sa_ref[...] + sb_ref[...] + hp_ref[...]) * dis + b_ref[...]
    out_ref[...] = jnp.dot(h, w_ref[...], preferred_element_type=jnp.float32) * dis


def _tc2(sa, sb, hp, dis, b, w):
    grid = (N // BR,)
    return pl.pallas_call(
        _tc2_body,
        grid=grid,
        in_specs=[
            pl.BlockSpec((BR, D), lambda i: (i, 0)),
            pl.BlockSpec((BR, D), lambda i: (i, 0)),
            pl.BlockSpec((BR, D), lambda i: (i, 0)),
            pl.BlockSpec((BR, 1), lambda i: (i, 0)),
            pl.BlockSpec((1, D), lambda i: (0, 0)),
            pl.BlockSpec((D, D), lambda i: (0, 0)),
        ],
        out_specs=pl.BlockSpec((BR, D), lambda i: (i, 0)),
        out_shape=jax.ShapeDtypeStruct((N, D), jnp.float32),
    )(sa, sb, hp, dis, b, w)


def _tc3_body(sa_ref, sb_ref, hp_ref, dis_ref, b_ref, out_ref):
    out_ref[...] = ((sa_ref[...] + sb_ref[...] + hp_ref[...]) * dis_ref[...]
                    + b_ref[...])


def _tc3(sa, sb, hp, dis, b):
    grid = (N // BR,)
    return pl.pallas_call(
        _tc3_body,
        grid=grid,
        in_specs=[
            pl.BlockSpec((BR, D), lambda i: (i, 0)),
            pl.BlockSpec((BR, D), lambda i: (i, 0)),
            pl.BlockSpec((BR, D), lambda i: (i, 0)),
            pl.BlockSpec((BR, 1), lambda i: (i, 0)),
            pl.BlockSpec((1, D), lambda i: (0, 0)),
        ],
        out_specs=pl.BlockSpec((BR, D), lambda i: (i, 0)),
        out_shape=jax.ShapeDtypeStruct((N, D), jnp.float32),
    )(sa, sb, hp, dis, b)


def kernel(x, edge_index, W1, b1, W2, b2):
    ei = edge_index.astype(jnp.int32)
    src, dst = ei[0], ei[1]
    E = src.shape[0]
    epad_a = NS * KA * GROUP
    epad_b = NS * KB * GROUP
    epad = epad_a + epad_b

    def split(v, fill_spread):
        lp = epad - E
        vf = jnp.concatenate([v, fill_spread[:lp]])
        va = vf[:epad_a].reshape(NS, KA, GROUP)
        vb = vf[epad_a:].reshape(NS, KB, GROUP)
        va = jnp.pad(va, ((0, 0), (0, KMAX - KA), (0, 0)))
        vb = jnp.pad(vb, ((0, 0), (0, KMAX - KB), (0, 0)))
        return jnp.concatenate([va, vb], axis=0)

    lp = epad - E
    pad_ids = jnp.arange(lp, dtype=jnp.int32)
    src_p = split(src, pad_ids % N)
    dst_p = split(dst, N + pad_ids % (R - N))

    m1 = _tcmm(x, W1)
    degp = _sc_degree(dst_p)
    dega = degp[0, :N].reshape(N, 1)
    degb = degp[1, :N].reshape(N, 1)
    dis, hp1 = _tc1(dega, degb, m1)

    s1 = _sc_scatter(hp1, src_p, dst_p)
    hp2 = _tc2(s1[0, :N], s1[1, :N], hp1, dis, b1.reshape(1, D), W2)

    s2 = _sc_scatter(hp2, src_p, dst_p)
    return _tc3(s2[0, :N], s2[1, :N], hp2, dis, b2.reshape(1, D))


# speedup vs baseline: 1.1157x; 1.1157x over previous
import jax
import jax.numpy as jnp
from jax import lax
from jax.experimental import pallas as pl
from jax.experimental.pallas import tpu as pltpu
from jax.experimental.pallas import tpu_sc as plsc

N = 10000
D = 128
NC = 2
NS = 16
NW = NC * NS
GROUP = 128
R = 10240
RPT = R // NS
BR = 1000

_MESH = dict(mesh=plsc.VectorSubcoreMesh(core_axis_name="c", subcore_axis_name="s"))


def _deg_body(dst_hbm, ones_hbm, zero_hbm, out_hbm, dst_v, ones_v, deg_sh, sem):
    c = lax.axis_index("c")
    s = lax.axis_index("s")
    wid = c * NS + s
    kc = KA if KA == KB else jnp.where(c == 0, KA, KB)
    pltpu.sync_copy(ones_hbm, ones_v)
    pltpu.sync_copy(zero_hbm, deg_sh.at[pl.ds(s * RPT, RPT)])
    pltpu.sync_copy(dst_hbm.at[wid], dst_v)
    plsc.subcore_barrier()

    def issue(j, carry):
        pltpu.async_copy(ones_v, deg_sh.at[dst_v.at[j]], sem, add=True)
        return carry

    lax.fori_loop(0, kc, issue, 0)

    def drain(j, carry):
        pltpu.make_async_copy(ones_v, deg_sh.at[dst_v.at[0]], sem).wait()
        return carry

    lax.fori_loop(0, kc, drain, 0)
    plsc.subcore_barrier()
    pltpu.sync_copy(deg_sh.at[pl.ds(s * RPT, RPT)], out_hbm.at[c, pl.ds(s * RPT, RPT)])


def _sc_degree(dst_p):
    ones = jnp.ones((GROUP,), jnp.float32)
    zero = jnp.zeros((RPT,), jnp.float32)
    return pl.kernel(
        _deg_body,
        out_type=jax.ShapeDtypeStruct((NC, R), jnp.float32),
        scratch_types=[
            pltpu.VMEM((KMAX, GROUP), jnp.int32),
            pltpu.VMEM((GROUP,), jnp.float32),
            pltpu.VMEM_SHARED((R,), jnp.float32),
            pltpu.SemaphoreType.DMA,
        ],
        **_MESH,
    )(dst_p, ones, zero)


KA = 80
KB = 80
KMAX = max(KA, KB)

CG = 16
NCHUNK = 5


def _scat_body(hp_hbm, src_hbm, dst_hbm, zeros_hbm, out_hbm,
               srcc, dstc, rows0, rows1, out_sh,
               isem, gsem0, gsem1, ssem0, ssem1):
    c = lax.axis_index("c")
    s = lax.axis_index("s")
    wid = c * NS + s
    i0 = pltpu.async_copy(src_hbm.at[wid, pl.ds(0, CG)], srcc.at[0], isem)
    i1 = pltpu.async_copy(dst_hbm.at[wid, pl.ds(0, CG)], dstc.at[0], isem)
    pltpu.sync_copy(zeros_hbm, out_sh.at[pl.ds(s * RPT, RPT)])
    i0.wait()
    i1.wait()
    plsc.subcore_barrier()

    for t in range(NCHUNK):
        cb = t % 2
        if t + 1 < NCHUNK:
            pltpu.async_copy(src_hbm.at[wid, pl.ds((t + 1) * CG, CG)],
                             srcc.at[1 - cb], isem)
            pltpu.async_copy(dst_hbm.at[wid, pl.ds((t + 1) * CG, CG)],
                             dstc.at[1 - cb], isem)

        pltpu.async_copy(hp_hbm.at[srcc.at[cb, 0]], rows0, gsem0)

        def pair(p, carry):
            pltpu.make_async_copy(hp_hbm.at[srcc.at[cb, 2 * p]], rows0,
                                  gsem0).wait()
            pltpu.async_copy(hp_hbm.at[srcc.at[cb, 2 * p + 1]], rows1, gsem1)
            s0 = pltpu.async_copy(rows0, out_sh.at[dstc.at[cb, 2 * p]],
                                  ssem0, add=True)
            s0.wait()
            nxt = lax.rem(2 * p + 2, CG)
            pltpu.async_copy(hp_hbm.at[srcc.at[cb, nxt]], rows0, gsem0)
            pltpu.make_async_copy(hp_hbm.at[srcc.at[cb, 2 * p + 1]], rows1,
                                  gsem1).wait()
            s1 = pltpu.async_copy(rows1, out_sh.at[dstc.at[cb, 2 * p + 1]],
                                  ssem1, add=True)
            s1.wait()
            return carry

        lax.fori_loop(0, CG // 2, pair, 0)
        pltpu.make_async_copy(hp_hbm.at[srcc.at[cb, 0]], rows0, gsem0).wait()
        if t + 1 < NCHUNK:
            pltpu.make_async_copy(src_hbm.at[wid, pl.ds(0, CG)],
                                  srcc.at[1 - cb], isem).wait()
            pltpu.make_async_copy(dst_hbm.at[wid, pl.ds(0, CG)],
                                  dstc.at[1 - cb], isem).wait()

    plsc.subcore_barrier()
    pltpu.sync_copy(out_sh.at[pl.ds(s * RPT, RPT)], out_hbm.at[c, pl.ds(s * RPT, RPT)])


def _sc_scatter(hp, src_p, dst_p):
    zeros = jnp.zeros((RPT, D), jnp.float32)
    return pl.kernel(
        _scat_body,
        out_type=jax.ShapeDtypeStruct((NC, R, D), jnp.float32),
        scratch_types=[
            pltpu.VMEM((2, CG, GROUP), jnp.int32),
            pltpu.VMEM((2, CG, GROUP), jnp.int32),
            pltpu.VMEM((GROUP, D), jnp.float32),
            pltpu.VMEM((GROUP, D), jnp.float32),
            pltpu.VMEM_SHARED((R, D), jnp.float32),
            pltpu.SemaphoreType.DMA,
            pltpu.SemaphoreType.DMA,
            pltpu.SemaphoreType.DMA,
            pltpu.SemaphoreType.DMA,
            pltpu.SemaphoreType.DMA,
        ],
        **_MESH,
    )(hp, src_p, dst_p, zeros)


def _tcmm_body(x_ref, w_ref, m_ref):
    m_ref[...] = jnp.dot(x_ref[...], w_ref[...],
                         preferred_element_type=jnp.float32)


def _tcmm(x, w):
    grid = (N // BR,)
    return pl.pallas_call(
        _tcmm_body,
        grid=grid,
        in_specs=[
            pl.BlockSpec((BR, D), lambda i: (i, 0)),
            pl.BlockSpec((D, D), lambda i: (0, 0)),
        ],
        out_specs=pl.BlockSpec((BR, D), lambda i: (i, 0)),
        out_shape=jax.ShapeDtypeStruct((N, D), jnp.float32),
    )(x, w)


def _tc1_body(dega_ref, degb_ref, m_ref, dis_ref, hp_ref):
    deg = dega_ref[...] + degb_ref[...] + 1.0
    dis = lax.rsqrt(deg)
    dis_ref[...] = dis
    hp_ref[...] = m_ref[...] * dis


def _tc1(dega, degb, m):
    grid = (N // BR,)
    return pl.pallas_call(
        _tc1_body,
        grid=grid,
        in_specs=[
            pl.BlockSpec((BR, 1), lambda i: (i, 0)),
            pl.BlockSpec((BR, 1), lambda i: (i, 0)),
            pl.BlockSpec((BR, D), lambda i: (i, 0)),
        ],
        out_specs=[
            pl.BlockSpec((BR, 1), lambda i: (i, 0)),
            pl.BlockSpec((BR, D), lambda i: (i, 0)),
        ],
        out_shape=[
            jax.ShapeDtypeStruct((N, 1), jnp.float32),
            jax.ShapeDtypeStruct((N, D), jnp.float32),
        ],
    )(dega, degb, m)


def _tc2_body(sa_ref, sb_ref, hp_ref, dis_ref, b_ref, w_ref, out_ref):
    dis = dis_ref[...]
    h = (sa_ref[...] + sb_ref[...] + hp_ref[...]) * dis + b_ref[...]
    out_ref[...] = jnp.dot(h, w_ref[...], preferred_element_type=jnp.float32) * dis


def _tc2(sa, sb, hp, dis, b, w):
    grid = (N // BR,)
    return pl.pallas_call(
        _tc2_body,
        grid=grid,
        in_specs=[
            pl.BlockSpec((BR, D), lambda i: (i, 0)),
            pl.BlockSpec((BR, D), lambda i: (i, 0)),
            pl.BlockSpec((BR, D), lambda i: (i, 0)),
            pl.BlockSpec((BR, 1), lambda i: (i, 0)),
            pl.BlockSpec((1, D), lambda i: (0, 0)),
            pl.BlockSpec((D, D), lambda i: (0, 0)),
        ],
        out_specs=pl.BlockSpec((BR, D), lambda i: (i, 0)),
        out_shape=jax.ShapeDtypeStruct((N, D), jnp.float32),
    )(sa, sb, hp, dis, b, w)


def _tc3_body(sa_ref, sb_ref, hp_ref, dis_ref, b_ref, out_ref):
    out_ref[...] = ((sa_ref[...] + sb_ref[...] + hp_ref[...]) * dis_ref[...]
                    + b_ref[...])


def _tc3(sa, sb, hp, dis, b):
    grid = (N // BR,)
    return pl.pallas_call(
        _tc3_body,
        grid=grid,
        in_specs=[
            pl.BlockSpec((BR, D), lambda i: (i, 0)),
            pl.BlockSpec((BR, D), lambda i: (i, 0)),
            pl.BlockSpec((BR, D), lambda i: (i, 0)),
            pl.BlockSpec((BR, 1), lambda i: (i, 0)),
            pl.BlockSpec((1, D), lambda i: (0, 0)),
        ],
        out_specs=pl.BlockSpec((BR, D), lambda i: (i, 0)),
        out_shape=jax.ShapeDtypeStruct((N, D), jnp.float32),
    )(sa, sb, hp, dis, b)


def kernel(x, edge_index, W1, b1, W2, b2):
    ei = edge_index.astype(jnp.int32)
    src, dst = ei[0], ei[1]
    E = src.shape[0]
    epad_a = NS * KA * GROUP
    epad_b = NS * KB * GROUP
    epad = epad_a + epad_b

    def split(v, fill_spread):
        lp = epad - E
        vf = jnp.concatenate([v, fill_spread[:lp]])
        va = vf[:epad_a].reshape(NS, KA, GROUP)
        vb = vf[epad_a:].reshape(NS, KB, GROUP)
        va = jnp.pad(va, ((0, 0), (0, KMAX - KA), (0, 0)))
        vb = jnp.pad(vb, ((0, 0), (0, KMAX - KB), (0, 0)))
        return jnp.concatenate([va, vb], axis=0)

    lp = epad - E
    pad_ids = jnp.arange(lp, dtype=jnp.int32)
    src_p = split(src, pad_ids % N)
    dst_p = split(dst, N + pad_ids % (R - N))

    m1 = _tcmm(x, W1)
    degp = _sc_degree(dst_p)
    dega = degp[0, :N].reshape(N, 1)
    degb = degp[1, :N].reshape(N, 1)
    dis, hp1 = _tc1(dega, degb, m1)

    s1 = _sc_scatter(hp1, src_p, dst_p)
    hp2 = _tc2(s1[0, :N], s1[1, :N], hp1, dis, b1.reshape(1, D), W2)

    s2 = _sc_scatter(hp2, src_p, dst_p)
    return _tc3(s2[0, :N], s2[1, :N], hp2, dis, b2.reshape(1, D))
